# baseline (speedup 1.0000x reference)
import jax
import jax.numpy as jnp
from jax import lax
from jax.experimental import pallas as pl
from jax.experimental.pallas import tpu as pltpu
from jax.experimental.pallas import tpu_sc as plsc

N = 100000
K = 17
F = 8
NW = 32

SBA = 250
NSBA = N // SBA
MA = (NSBA + NW - 1) // NW

RB = 128
NBF = N // RB
TAIL = N - NBF * RB
NBT = NBF + 1
MB = (NBT + NW - 1) // NW
IDXB = RB * K
CH = IDXB // 128

_SC_PARAMS = pltpu.CompilerParams(
    use_tc_tiling_on_sc=False, needs_layout_passes=False)


def _sqrt16(x):
    xi = plsc.bitcast(x, jnp.int32)
    y = plsc.bitcast(jnp.int32(0x5F3759DF) - (xi >> 1), jnp.float32)
    for _ in range(3):
        y = y * (1.5 - 0.5 * x * y * y)
    return x * y


def _feat_body(tri_hbm, bary_hbm, feat_hbm, tri_v, bary_v, feat_v):
    wid = lax.axis_index("c") * 16 + lax.axis_index("s")

    def sub_batch(m, carry):
        b = m * NW + wid

        @pl.when(b < NSBA)
        def _():
            r0 = b * SBA
            pltpu.sync_copy(tri_hbm.at[pl.ds(r0, SBA)], tri_v)
            pltpu.sync_copy(bary_hbm.at[pl.ds(r0, SBA)], bary_v)
            lane = lax.iota(jnp.int32, 16)
            for g in range(16):
                mask = None if g < 15 else lane < (SBA - 15 * 16)
                rows = g * 16 + lane
                vg = [plsc.load_gather(
                          tri_v, [rows, jnp.full((16,), c // 3, jnp.int32),
                                  jnp.full((16,), c % 3, jnp.int32)],
                          mask=mask)
                      for c in range(9)]
                s = []
                for (a, b2) in ((0, 3), (0, 6), (3, 6)):
                    dx = vg[a] - vg[b2]
                    dy = vg[a + 1] - vg[b2 + 1]
                    dz = vg[a + 2] - vg[b2 + 2]
                    s.append(dx * dx + dy * dy + dz * dz)
                mn = _sqrt16(jnp.minimum(jnp.minimum(s[0], s[1]), s[2]))
                mx = _sqrt16(jnp.maximum(jnp.maximum(s[0], s[1]), s[2]))
                cols = [mn, mx]
                for c in range(3):
                    cols.append(plsc.load_gather(
                        bary_v, [rows, jnp.full((16,), c, jnp.int32)],
                        mask=mask))
                for c in range(5):
                    plsc.store_scatter(
                        feat_v, [rows, jnp.full((16,), c, jnp.int32)],
                        cols[c], mask=mask)
            pltpu.sync_copy(feat_v, feat_hbm.at[pl.ds(r0, SBA)])
        return carry

    lax.fori_loop(0, MA, sub_batch, 0)


def _feat_sc(tri, bary):
    mesh = plsc.VectorSubcoreMesh(core_axis_name="c", subcore_axis_name="s")
    return pl.kernel(
        _feat_body,
        out_type=jax.ShapeDtypeStruct((N, F), jnp.float32),
        mesh=mesh,
        scratch_types=[
            pltpu.VMEM((SBA, 3, 3), jnp.float32),
            pltpu.VMEM((SBA, 3), jnp.float32),
            pltpu.VMEM((SBA, F), jnp.float32),
        ],
        compiler_params=_SC_PARAMS,
    )(tri, bary)


def _rmat_body(feat_hbm, idx_hbm, out_hbm, idx0, idx1, g0, g1, o_v, gs0, gs1):
    wid = lax.axis_index("c") * 16 + lax.axis_index("s")

    def b_of(t):
        return t * NW + wid

    def stage_and_fire(b, idx_v, g_v, gsem):
        @pl.when(b < NBF)
        def _():
            pltpu.sync_copy(idx_hbm.at[pl.ds(b * IDXB, IDXB)], idx_v)

        @pl.when(b == NBF)
        def _():
            pltpu.sync_copy(idx_hbm.at[pl.ds(NBF * IDXB, TAIL * K)],
                            idx_v.at[pl.ds(0, TAIL * K)])
        for c in range(CH):
            pltpu.async_copy(feat_hbm.at[idx_v.at[pl.ds(c * 128, 128)]],
                             g_v.at[pl.ds(c * 128, 128)], gsem)

    def compute(b, idx_v, g_v, gsem):
        del idx_v
        pltpu.make_async_copy(feat_hbm.at[pl.ds(0, IDXB)], g_v, gsem).wait()
        nrows = jnp.where(b == NBF, TAIL, RB)

        def row_body(i, carry3):
            lane = lax.iota(jnp.int32, 16)
            ivec = jnp.full((16,), i, jnp.int32)
            bvec = jnp.full((16,), i * K, jnp.int32)
            for t in range(5):
                p = t * 16 + lane
                jr = p // 5
                jc = p % 5
                cvals = plsc.load_gather(g_v, [bvec, jc])
                nvals = plsc.load_gather(g_v, [bvec + 1 + jr, jc])
                plsc.store_scatter(o_v, [ivec, jr, jc], cvals - nvals)
            return carry3
        lax.fori_loop(0, nrows, row_body, 0)

        @pl.when(b < NBF)
        def _():
            pltpu.sync_copy(o_v, out_hbm.at[pl.ds(b * RB, RB)])

        @pl.when(b == NBF)
        def _():
            pltpu.sync_copy(o_v.at[pl.ds(0, TAIL)],
                            out_hbm.at[pl.ds(NBF * RB, TAIL)])

    def guarded(t, fn, *args):
        @pl.when(b_of(t) < NBT)
        def _():
            fn(b_of(t), *args)

    guarded(0, stage_and_fire, idx0, g0, gs0)

    def pair_body(q, carry):
        tA = 2 * q
        guarded(tA + 1, stage_and_fire, idx1, g1, gs1)
        guarded(tA, compute, idx0, g0, gs0)
        guarded(tA + 2, stage_and_fire, idx0, g0, gs0)
        guarded(tA + 1, compute, idx1, g1, gs1)
        return carry

    lax.fori_loop(0, (MB + 1) // 2, pair_body, 0)


def _rmatrix_sc(feat, idx):
    mesh = plsc.VectorSubcoreMesh(core_axis_name="c", subcore_axis_name="s")
    return pl.kernel(
        _rmat_body,
        out_type=jax.ShapeDtypeStruct((N, K - 1, 5), jnp.float32),
        mesh=mesh,
        scratch_types=[
            pltpu.VMEM((IDXB,), jnp.int32),
            pltpu.VMEM((IDXB,), jnp.int32),
            pltpu.VMEM((IDXB, F), jnp.float32),
            pltpu.VMEM((IDXB, F), jnp.float32),
            pltpu.VMEM((RB, K - 1, 5), jnp.float32),
            pltpu.SemaphoreType.DMA,
            pltpu.SemaphoreType.DMA,
        ],
        compiler_params=_SC_PARAMS,
    )(feat, idx)


def kernel(triangles, barycenters, indices_neigh_tri, number_neigh_tri):
    del number_neigh_tri
    idx32 = indices_neigh_tri.astype(jnp.int32).reshape(N * K)
    feat = _feat_sc(triangles, barycenters)
    return _rmatrix_sc(feat, idx32)


# speedup vs baseline: 1.5248x; 1.5248x over previous
import jax
import jax.numpy as jnp
from jax import lax
from jax.experimental import pallas as pl
from jax.experimental.pallas import tpu as pltpu
from jax.experimental.pallas import tpu_sc as plsc

N = 100000
K = 17
F = 8
NW = 32

SBA = 250
NSBA = N // SBA
MA = (NSBA + NW - 1) // NW

RB = 128
NBF = N // RB
TAIL = N - NBF * RB
NBT = NBF + 1
MB = (NBT + NW - 1) // NW
IDXB = RB * K
CH = IDXB // 128

_SC_PARAMS = pltpu.CompilerParams(
    use_tc_tiling_on_sc=False, needs_layout_passes=False)


def _sqrt16(x):
    xi = plsc.bitcast(x, jnp.int32)
    y = plsc.bitcast(jnp.int32(0x5F3759DF) - (xi >> 1), jnp.float32)
    for _ in range(3):
        y = y * (1.5 - 0.5 * x * y * y)
    return x * y


def _feat_body(tri_hbm, bary_hbm, feat_hbm, tri_v, bary_v, feat_v):
    wid = lax.axis_index("c") * 16 + lax.axis_index("s")

    def sub_batch(m, carry):
        b = m * NW + wid

        @pl.when(b < NSBA)
        def _():
            r0 = b * SBA
            pltpu.sync_copy(tri_hbm.at[pl.ds(r0, SBA)], tri_v)
            pltpu.sync_copy(bary_hbm.at[pl.ds(r0, SBA)], bary_v)
            lane = lax.iota(jnp.int32, 16)
            for g in range(16):
                mask = None if g < 15 else lane < (SBA - 15 * 16)
                rows = g * 16 + lane
                vg = [plsc.load_gather(
                          tri_v, [rows, jnp.full((16,), c // 3, jnp.int32),
                                  jnp.full((16,), c % 3, jnp.int32)],
                          mask=mask)
                      for c in range(9)]
                s = []
                for (a, b2) in ((0, 3), (0, 6), (3, 6)):
                    dx = vg[a] - vg[b2]
                    dy = vg[a + 1] - vg[b2 + 1]
                    dz = vg[a + 2] - vg[b2 + 2]
                    s.append(dx * dx + dy * dy + dz * dz)
                mn = _sqrt16(jnp.minimum(jnp.minimum(s[0], s[1]), s[2]))
                mx = _sqrt16(jnp.maximum(jnp.maximum(s[0], s[1]), s[2]))
                cols = [mn, mx]
                for c in range(3):
                    cols.append(plsc.load_gather(
                        bary_v, [rows, jnp.full((16,), c, jnp.int32)],
                        mask=mask))
                for c in range(5):
                    plsc.store_scatter(
                        feat_v, [rows, jnp.full((16,), c, jnp.int32)],
                        cols[c], mask=mask)
            pltpu.sync_copy(feat_v, feat_hbm.at[pl.ds(r0, SBA)])
        return carry

    lax.fori_loop(0, MA, sub_batch, 0)


def _feat_sc(tri, bary):
    mesh = plsc.VectorSubcoreMesh(core_axis_name="c", subcore_axis_name="s")
    return pl.kernel(
        _feat_body,
        out_type=jax.ShapeDtypeStruct((N, F), jnp.float32),
        mesh=mesh,
        scratch_types=[
            pltpu.VMEM((SBA, 3, 3), jnp.float32),
            pltpu.VMEM((SBA, 3), jnp.float32),
            pltpu.VMEM((SBA, F), jnp.float32),
        ],
        compiler_params=_SC_PARAMS,
    )(tri, bary)


def _rmat_body(feat_hbm, idx_hbm, out_hbm,
               idx2d0, idx2d1, idx0, idx1, g0, g1, o_v, gs0, gs1):
    wid = lax.axis_index("c") * 16 + lax.axis_index("s")

    def b_of(t):
        return t * NW + wid

    def stage_and_fire(b, idx2d_v, idx_v, g_v, gsem):
        @pl.when(b < NBF)
        def _():
            pltpu.sync_copy(idx_hbm.at[pl.ds(b * RB, RB)], idx2d_v)

        @pl.when(b == NBF)
        def _():
            pltpu.sync_copy(idx_hbm.at[pl.ds(NBF * RB, TAIL)],
                            idx2d_v.at[pl.ds(0, TAIL)])
        lane = lax.iota(jnp.int32, 16)
        for u in range(IDXB // 16):
            q0 = u * 16
            col = (q0 % K) + lane
            over = (col >= K).astype(jnp.int32)
            row = (q0 // K) + over
            col = col - K * over
            idx_v[pl.ds(q0, 16)] = plsc.load_gather(idx2d_v, [row, col])
        for c in range(CH):
            pltpu.async_copy(feat_hbm.at[idx_v.at[pl.ds(c * 128, 128)]],
                             g_v.at[pl.ds(c * 128, 128)], gsem)

    def compute(b, idx_v, g_v, gsem):
        del idx_v
        pltpu.make_async_copy(feat_hbm.at[pl.ds(0, IDXB)], g_v, gsem).wait()
        nrows = jnp.where(b == NBF, TAIL, RB)

        def row_body(i, carry3):
            lane = lax.iota(jnp.int32, 16)
            bvec = jnp.full((16,), i * K, jnp.int32)
            for t in range(5):
                p = t * 16 + lane
                jr = p // 5
                jc = p % 5
                cvals = plsc.load_gather(g_v, [bvec, jc])
                nvals = plsc.load_gather(g_v, [bvec + 1 + jr, jc])
                o_v[i, pl.ds(t * 16, 16)] = cvals - nvals
            return carry3
        lax.fori_loop(0, nrows, row_body, 0)

        @pl.when(b < NBF)
        def _():
            pltpu.sync_copy(o_v, out_hbm.at[pl.ds(b * RB, RB)])

        @pl.when(b == NBF)
        def _():
            pltpu.sync_copy(o_v.at[pl.ds(0, TAIL)],
                            out_hbm.at[pl.ds(NBF * RB, TAIL)])

    def guarded(t, fn, *args):
        @pl.when(b_of(t) < NBT)
        def _():
            fn(b_of(t), *args)

    guarded(0, stage_and_fire, idx2d0, idx0, g0, gs0)

    def pair_body(q, carry):
        tA = 2 * q
        guarded(tA + 1, stage_and_fire, idx2d1, idx1, g1, gs1)
        guarded(tA, compute, idx0, g0, gs0)
        guarded(tA + 2, stage_and_fire, idx2d0, idx0, g0, gs0)
        guarded(tA + 1, compute, idx1, g1, gs1)
        return carry

    lax.fori_loop(0, (MB + 1) // 2, pair_body, 0)


def _rmatrix_sc(feat, idx):
    mesh = plsc.VectorSubcoreMesh(core_axis_name="c", subcore_axis_name="s")
    return pl.kernel(
        _rmat_body,
        out_type=jax.ShapeDtypeStruct((N, (K - 1) * 5), jnp.float32),
        mesh=mesh,
        scratch_types=[
            pltpu.VMEM((RB, K), jnp.int32),
            pltpu.VMEM((RB, K), jnp.int32),
            pltpu.VMEM((IDXB,), jnp.int32),
            pltpu.VMEM((IDXB,), jnp.int32),
            pltpu.VMEM((IDXB, F), jnp.float32),
            pltpu.VMEM((IDXB, F), jnp.float32),
            pltpu.VMEM((RB, (K - 1) * 5), jnp.float32),
            pltpu.SemaphoreType.DMA,
            pltpu.SemaphoreType.DMA,
        ],
        compiler_params=_SC_PARAMS,
    )(feat, idx)


def kernel(triangles, barycenters, indices_neigh_tri, number_neigh_tri):
    del number_neigh_tri
    idx32 = indices_neigh_tri.astype(jnp.int32)
    feat = _feat_sc(triangles, barycenters)
    out80 = _rmatrix_sc(feat, idx32)
    return out80.reshape(N, K - 1, 5)
